# baseline (device time: 83848 ns/iter reference)
import jax
import jax.numpy as jnp
from jax import lax
from jax.experimental import pallas as pl
from jax.experimental.pallas import tpu as pltpu

N_DEV = 16
N_TOK = 512
D_OUT = 512
E_PER = 2
CHUNK = N_TOK // N_DEV
N_HOP = N_DEV - 1


def kernel(x, router_W, route_idx, expert_W):
    del router_W

    def body(x_ref, idx_ref, w_ref, out_ref, acc_ref, rs_buf,
             rs_send, rs_recv, ag_send, ag_recv):
        my = lax.axis_index("i")
        left = (my - 1) % N_DEV
        right = (my + 1) % N_DEV

        barrier = pltpu.get_barrier_semaphore()
        for nbr in (left, right):
            pl.semaphore_signal(barrier, inc=1, device_id=(nbr,),
                                device_id_type=pl.DeviceIdType.MESH)
        pl.semaphore_wait(barrier, 2)

        route = idx_ref[:, :]
        x_all = x_ref[:, :]
        e0 = my * E_PER
        partial = jnp.zeros((N_TOK, D_OUT), jnp.float32)
        for k in range(E_PER):
            mask = (route == (e0 + k)).astype(jnp.float32)
            partial = partial + jnp.dot(
                x_all * mask, w_ref[k], preferred_element_type=jnp.float32)
        acc_ref[:, :] = partial

        for h in range(N_HOP):
            s = (my - h) % N_DEV
            rdma = pltpu.make_async_remote_copy(
                src_ref=acc_ref.at[pl.ds(s * CHUNK, CHUNK), :],
                dst_ref=rs_buf.at[h],
                send_sem=rs_send.at[h],
                recv_sem=rs_recv.at[h],
                device_id=(right,),
                device_id_type=pl.DeviceIdType.MESH,
            )
            rdma.start()
            rdma.wait()
            r = (my - h - 1) % N_DEV
            acc_ref[pl.ds(r * CHUNK, CHUNK), :] = (
                acc_ref[pl.ds(r * CHUNK, CHUNK), :] + rs_buf[h])

        for h in range(N_HOP):
            s = (my + 1 - h) % N_DEV
            rdma = pltpu.make_async_remote_copy(
                src_ref=acc_ref.at[pl.ds(s * CHUNK, CHUNK), :],
                dst_ref=acc_ref.at[pl.ds(s * CHUNK, CHUNK), :],
                send_sem=ag_send.at[h],
                recv_sem=ag_recv.at[h],
                device_id=(right,),
                device_id_type=pl.DeviceIdType.MESH,
            )
            rdma.start()
            rdma.wait()

        out_ref[:, :] = acc_ref[:, :]

    return pl.pallas_call(
        body,
        out_shape=jax.ShapeDtypeStruct((N_TOK, D_OUT), jnp.float32),
        in_specs=[
            pl.BlockSpec(memory_space=pltpu.VMEM),
            pl.BlockSpec(memory_space=pltpu.VMEM),
            pl.BlockSpec(memory_space=pltpu.VMEM),
        ],
        out_specs=pl.BlockSpec(memory_space=pltpu.VMEM),
        scratch_shapes=[
            pltpu.VMEM((N_TOK, D_OUT), jnp.float32),
            pltpu.VMEM((N_HOP, CHUNK, D_OUT), jnp.float32),
            pltpu.SemaphoreType.DMA((N_HOP,)),
            pltpu.SemaphoreType.DMA((N_HOP,)),
            pltpu.SemaphoreType.DMA((N_HOP,)),
            pltpu.SemaphoreType.DMA((N_HOP,)),
        ],
        compiler_params=pltpu.CompilerParams(collective_id=0),
    )(x, route_idx, expert_W)


# device time: 55315 ns/iter; 1.5158x vs baseline; 1.5158x over previous
import jax
import jax.numpy as jnp
from jax import lax
from jax.experimental import pallas as pl
from jax.experimental.pallas import tpu as pltpu

N_DEV = 16
N_TOK = 512
D_OUT = 512
E_PER = 2
CHUNK = N_TOK // N_DEV
N_STEP = 4

BIT_ORDER = (3, 2, 1, 0)

RS_ROWS = [(1 << (3 - t)) * CHUNK for t in range(N_STEP)]
RS_OFF = [sum(RS_ROWS[:t]) for t in range(N_STEP)]
BUF_ROWS = sum(RS_ROWS)


def _virt(d):
    v = 0
    for t, k in enumerate(BIT_ORDER):
        v = v + ((d >> k) & 1) * (1 << (3 - t))
    return v


_PERM = [_virt(c) for c in range(N_DEV)]


def kernel(x, router_W, route_idx, expert_W):
    del router_W

    def body(x_ref, idx_ref, w_ref, out_ref, acc_ref, bf_buf,
             rs_send, rs_recv, ag_send, ag_recv):
        my = lax.axis_index("i")
        v = _virt(my)

        barrier = pltpu.get_barrier_semaphore()
        for b in range(N_STEP):
            pl.semaphore_signal(barrier, inc=1, device_id=(my ^ (1 << b),),
                                device_id_type=pl.DeviceIdType.MESH)
        pl.semaphore_wait(barrier, N_STEP)

        route = idx_ref[:, :]
        x_all = x_ref[:, :]
        e0 = my * E_PER
        partial = jnp.zeros((N_TOK, D_OUT), jnp.float32)
        for k in range(E_PER):
            mask = (route == (e0 + k)).astype(jnp.float32)
            partial = partial + jnp.dot(
                x_all * mask, w_ref[k], preferred_element_type=jnp.float32)

        if BIT_ORDER == (3, 2, 1, 0):
            acc_ref[:, :] = partial
        else:
            for c in range(N_DEV):
                acc_ref[pl.ds(_PERM[c] * CHUNK, CHUNK), :] = lax.slice(
                    partial, (c * CHUNK, 0), ((c + 1) * CHUNK, D_OUT))

        for t in range(N_STEP):
            vb = 3 - t
            partner = my ^ (1 << BIT_ORDER[t])
            rows = RS_ROWS[t]
            keep = ((v >> vb) << vb) * CHUNK
            send = (((v >> vb) << vb) ^ (1 << vb)) * CHUNK
            rdma = pltpu.make_async_remote_copy(
                src_ref=acc_ref.at[pl.ds(send, rows), :],
                dst_ref=bf_buf.at[pl.ds(RS_OFF[t], rows), :],
                send_sem=rs_send.at[t],
                recv_sem=rs_recv.at[t],
                device_id=(partner,),
                device_id_type=pl.DeviceIdType.MESH,
            )
            rdma.start()
            rdma.wait()
            acc_ref[pl.ds(keep, rows), :] = (
                acc_ref[pl.ds(keep, rows), :]
                + bf_buf[pl.ds(RS_OFF[t], rows), :])

        for u in range(N_STEP):
            partner = my ^ (1 << BIT_ORDER[N_STEP - 1 - u])
            rows = (1 << u) * CHUNK
            send = ((v >> u) << u) * CHUNK
            rdma = pltpu.make_async_remote_copy(
                src_ref=acc_ref.at[pl.ds(send, rows), :],
                dst_ref=acc_ref.at[pl.ds(send, rows), :],
                send_sem=ag_send.at[u],
                recv_sem=ag_recv.at[u],
                device_id=(partner,),
                device_id_type=pl.DeviceIdType.MESH,
            )
            rdma.start()
            rdma.wait()

        if BIT_ORDER == (3, 2, 1, 0):
            out_ref[:, :] = acc_ref[:, :]
        else:
            for c in range(N_DEV):
                out_ref[pl.ds(c * CHUNK, CHUNK), :] = acc_ref[
                    pl.ds(_PERM[c] * CHUNK, CHUNK), :]

    return pl.pallas_call(
        body,
        out_shape=jax.ShapeDtypeStruct((N_TOK, D_OUT), jnp.float32),
        in_specs=[
            pl.BlockSpec(memory_space=pltpu.VMEM),
            pl.BlockSpec(memory_space=pltpu.VMEM),
            pl.BlockSpec(memory_space=pltpu.VMEM),
        ],
        out_specs=pl.BlockSpec(memory_space=pltpu.VMEM),
        scratch_shapes=[
            pltpu.VMEM((N_TOK, D_OUT), jnp.float32),
            pltpu.VMEM((BUF_ROWS, D_OUT), jnp.float32),
            pltpu.SemaphoreType.DMA((N_STEP,)),
            pltpu.SemaphoreType.DMA((N_STEP,)),
            pltpu.SemaphoreType.DMA((N_STEP,)),
            pltpu.SemaphoreType.DMA((N_STEP,)),
        ],
        compiler_params=pltpu.CompilerParams(collective_id=0),
    )(x, route_idx, expert_W)


# device time: 44751 ns/iter; 1.8737x vs baseline; 1.2361x over previous
import jax
import jax.numpy as jnp
from jax import lax
from jax.experimental import pallas as pl
from jax.experimental.pallas import tpu as pltpu

N_DEV = 16
N_TOK = 512
D_OUT = 512
E_PER = 2
CHUNK = N_TOK // N_DEV
N_STEP = 4

BIT_ORDER = (0, 2, 1, 3)

RS_ROWS = [(1 << (3 - t)) * CHUNK for t in range(N_STEP)]
RS_OFF = [sum(RS_ROWS[:t]) for t in range(N_STEP)]
BUF_ROWS = sum(RS_ROWS)


def _virt(d):
    v = 0
    for t, k in enumerate(BIT_ORDER):
        v = v + ((d >> k) & 1) * (1 << (3 - t))
    return v


_PERM = [_virt(c) for c in range(N_DEV)]


def kernel(x, router_W, route_idx, expert_W):
    del router_W

    def body(x_ref, idx_ref, w_ref, out_ref, acc_ref, bf_buf,
             rs_send, rs_recv, ag_send, ag_recv):
        my = lax.axis_index("i")
        v = _virt(my)

        barrier = pltpu.get_barrier_semaphore()
        for b in range(N_STEP):
            pl.semaphore_signal(barrier, inc=1, device_id=(my ^ (1 << b),),
                                device_id_type=pl.DeviceIdType.MESH)
        pl.semaphore_wait(barrier, N_STEP)

        route = idx_ref[:, :]
        x_all = x_ref[:, :]
        e0 = my * E_PER
        partial = jnp.zeros((N_TOK, D_OUT), jnp.float32)
        for k in range(E_PER):
            mask = (route == (e0 + k)).astype(jnp.float32)
            partial = partial + jnp.dot(
                x_all * mask, w_ref[k], preferred_element_type=jnp.float32)

        if BIT_ORDER == (3, 2, 1, 0):
            acc_ref[:, :] = partial
        else:
            for c in range(N_DEV):
                acc_ref[pl.ds(_PERM[c] * CHUNK, CHUNK), :] = lax.slice(
                    partial, (c * CHUNK, 0), ((c + 1) * CHUNK, D_OUT))

        for t in range(N_STEP):
            vb = 3 - t
            partner = my ^ (1 << BIT_ORDER[t])
            rows = RS_ROWS[t]
            keep = ((v >> vb) << vb) * CHUNK
            send = (((v >> vb) << vb) ^ (1 << vb)) * CHUNK
            rdma = pltpu.make_async_remote_copy(
                src_ref=acc_ref.at[pl.ds(send, rows), :],
                dst_ref=bf_buf.at[pl.ds(RS_OFF[t], rows), :],
                send_sem=rs_send.at[t],
                recv_sem=rs_recv.at[t],
                device_id=(partner,),
                device_id_type=pl.DeviceIdType.MESH,
            )
            rdma.start()
            rdma.wait()
            acc_ref[pl.ds(keep, rows), :] = (
                acc_ref[pl.ds(keep, rows), :]
                + bf_buf[pl.ds(RS_OFF[t], rows), :])

        for u in range(N_STEP):
            partner = my ^ (1 << BIT_ORDER[N_STEP - 1 - u])
            rows = (1 << u) * CHUNK
            send = ((v >> u) << u) * CHUNK
            rdma = pltpu.make_async_remote_copy(
                src_ref=acc_ref.at[pl.ds(send, rows), :],
                dst_ref=acc_ref.at[pl.ds(send, rows), :],
                send_sem=ag_send.at[u],
                recv_sem=ag_recv.at[u],
                device_id=(partner,),
                device_id_type=pl.DeviceIdType.MESH,
            )
            rdma.start()
            rdma.wait()

        if BIT_ORDER == (3, 2, 1, 0):
            out_ref[:, :] = acc_ref[:, :]
        else:
            for c in range(N_DEV):
                out_ref[pl.ds(c * CHUNK, CHUNK), :] = acc_ref[
                    pl.ds(_PERM[c] * CHUNK, CHUNK), :]

    return pl.pallas_call(
        body,
        out_shape=jax.ShapeDtypeStruct((N_TOK, D_OUT), jnp.float32),
        in_specs=[
            pl.BlockSpec(memory_space=pltpu.VMEM),
            pl.BlockSpec(memory_space=pltpu.VMEM),
            pl.BlockSpec(memory_space=pltpu.VMEM),
        ],
        out_specs=pl.BlockSpec(memory_space=pltpu.VMEM),
        scratch_shapes=[
            pltpu.VMEM((N_TOK, D_OUT), jnp.float32),
            pltpu.VMEM((BUF_ROWS, D_OUT), jnp.float32),
            pltpu.SemaphoreType.DMA((N_STEP,)),
            pltpu.SemaphoreType.DMA((N_STEP,)),
            pltpu.SemaphoreType.DMA((N_STEP,)),
            pltpu.SemaphoreType.DMA((N_STEP,)),
        ],
        compiler_params=pltpu.CompilerParams(collective_id=0),
    )(x, route_idx, expert_W)


# device time: 34372 ns/iter; 2.4394x vs baseline; 1.3020x over previous
import jax
import jax.numpy as jnp
from jax import lax
from jax.experimental import pallas as pl
from jax.experimental.pallas import tpu as pltpu

N_DEV = 16
N_TOK = 512
D_OUT = 512
E_PER = 2
CHUNK = N_TOK // N_DEV
N_STEP = 4
N_FLOW = 2
COL_H = D_OUT // N_FLOW

ORDERS = ((0, 2, 1, 3), (2, 0, 3, 1))

RS_ROWS = [(1 << (N_STEP - 1 - t)) * CHUNK for t in range(N_STEP)]
RS_OFF = [sum(RS_ROWS[:t]) for t in range(N_STEP)]
BUF_ROWS = sum(RS_ROWS)


def _virt(d, order):
    v = 0
    for t, k in enumerate(order):
        v = v + ((d >> k) & 1) * (1 << (N_STEP - 1 - t))
    return v


_PERMS = [[_virt(c, o) for c in range(N_DEV)] for o in ORDERS]


def kernel(x, router_W, route_idx, expert_W):
    del router_W

    def body(x_ref, idx_ref, w_ref, out_ref, acc_ref, bf_buf,
             rs_send, rs_recv, ag_send, ag_recv):
        my = lax.axis_index("i")
        virt = [_virt(my, o) for o in ORDERS]

        barrier = pltpu.get_barrier_semaphore()
        for b in range(N_STEP):
            pl.semaphore_signal(barrier, inc=1, device_id=(my ^ (1 << b),),
                                device_id_type=pl.DeviceIdType.MESH)
        pl.semaphore_wait(barrier, N_STEP)

        route = idx_ref[:, :]
        x_all = x_ref[:, :]
        e0 = my * E_PER
        partial = jnp.zeros((N_TOK, D_OUT), jnp.float32)
        for k in range(E_PER):
            mask = (route == (e0 + k)).astype(jnp.float32)
            partial = partial + jnp.dot(
                x_all * mask, w_ref[k], preferred_element_type=jnp.float32)

        for f in range(N_FLOW):
            for c in range(N_DEV):
                acc_ref[f, pl.ds(_PERMS[f][c] * CHUNK, CHUNK), :] = lax.slice(
                    partial, (c * CHUNK, f * COL_H),
                    ((c + 1) * CHUNK, (f + 1) * COL_H))

        for t in range(N_STEP):
            vb = N_STEP - 1 - t
            rows = RS_ROWS[t]
            rdmas = []
            for f in range(N_FLOW):
                send = (((virt[f] >> vb) << vb) ^ (1 << vb)) * CHUNK
                rdma = pltpu.make_async_remote_copy(
                    src_ref=acc_ref.at[f, pl.ds(send, rows), :],
                    dst_ref=bf_buf.at[f, pl.ds(RS_OFF[t], rows), :],
                    send_sem=rs_send.at[t, f],
                    recv_sem=rs_recv.at[t, f],
                    device_id=(my ^ (1 << ORDERS[f][t]),),
                    device_id_type=pl.DeviceIdType.MESH,
                )
                rdma.start()
                rdmas.append(rdma)
            for f in range(N_FLOW):
                rdmas[f].wait()
                keep = ((virt[f] >> vb) << vb) * CHUNK
                acc_ref[f, pl.ds(keep, rows), :] = (
                    acc_ref[f, pl.ds(keep, rows), :]
                    + bf_buf[f, pl.ds(RS_OFF[t], rows), :])

        for u in range(N_STEP):
            rows = (1 << u) * CHUNK
            rdmas = []
            for f in range(N_FLOW):
                send = ((virt[f] >> u) << u) * CHUNK
                rdma = pltpu.make_async_remote_copy(
                    src_ref=acc_ref.at[f, pl.ds(send, rows), :],
                    dst_ref=acc_ref.at[f, pl.ds(send, rows), :],
                    send_sem=ag_send.at[u, f],
                    recv_sem=ag_recv.at[u, f],
                    device_id=(my ^ (1 << ORDERS[f][N_STEP - 1 - u]),),
                    device_id_type=pl.DeviceIdType.MESH,
                )
                rdma.start()
                rdmas.append(rdma)
            for rdma in rdmas:
                rdma.wait()

        for f in range(N_FLOW):
            for c in range(N_DEV):
                out_ref[pl.ds(c * CHUNK, CHUNK),
                        pl.ds(f * COL_H, COL_H)] = acc_ref[
                    f, pl.ds(_PERMS[f][c] * CHUNK, CHUNK), :]

    return pl.pallas_call(
        body,
        out_shape=jax.ShapeDtypeStruct((N_TOK, D_OUT), jnp.float32),
        in_specs=[
            pl.BlockSpec(memory_space=pltpu.VMEM),
            pl.BlockSpec(memory_space=pltpu.VMEM),
            pl.BlockSpec(memory_space=pltpu.VMEM),
        ],
        out_specs=pl.BlockSpec(memory_space=pltpu.VMEM),
        scratch_shapes=[
            pltpu.VMEM((N_FLOW, N_TOK, COL_H), jnp.float32),
            pltpu.VMEM((N_FLOW, BUF_ROWS, COL_H), jnp.float32),
            pltpu.SemaphoreType.DMA((N_STEP, N_FLOW)),
            pltpu.SemaphoreType.DMA((N_STEP, N_FLOW)),
            pltpu.SemaphoreType.DMA((N_STEP, N_FLOW)),
            pltpu.SemaphoreType.DMA((N_STEP, N_FLOW)),
        ],
        compiler_params=pltpu.CompilerParams(collective_id=0),
    )(x, route_idx, expert_W)
